# baseline (speedup 1.0000x reference)
import functools

import jax
import jax.numpy as jnp
from jax import lax
from jax.experimental import pallas as pl
from jax.experimental.pallas import tpu as pltpu
from jax.experimental.pallas import tpu_sc as plsc

_V = 1000000
_D = 64
_B = 4096
_L = 200
_NC = 2
_NS = 16
_NW = _NC * _NS
_BW = 128

_mesh = plsc.VectorSubcoreMesh(core_axis_name="c", subcore_axis_name="s")


@functools.partial(
    pl.kernel,
    mesh=_mesh,
    compiler_params=pltpu.CompilerParams(
        use_tc_tiling_on_sc=False, needs_layout_passes=False
    ),
    out_type=jax.ShapeDtypeStruct((_L, 8, _NW, 8, 128), jnp.float32),
    scratch_types=[
        pltpu.VMEM((_L, _BW), jnp.int32),
        [pltpu.VMEM((_BW, 2 * _D), jnp.float32) for _ in range(4)],
        [pltpu.VMEM((_D, 129), jnp.float32) for _ in range(2)],
        [pltpu.SemaphoreType.DMA for _ in range(4)],
        [pltpu.SemaphoreType.DMA for _ in range(2)],
    ],
)
def _embedding_gather(w_hbm, idx_hbm, out_hbm, idx_v, gbufs, obufs, gsems, osems):
    wid = lax.axis_index("s") * _NC + lax.axis_index("c")

    pltpu.sync_copy(idx_hbm.at[:, pl.ds(wid * _BW, _BW)], idx_v)

    def gather_cp(l, p):
        return pltpu.make_async_copy(w_hbm.at[idx_v.at[l]], gbufs[p], gsems[p])

    def out_cp(l, p, di):
        return pltpu.make_async_copy(
            obufs[p].at[pl.ds(di * 8, 8), pl.ds(0, 128)],
            out_hbm.at[l, di, wid],
            osems[p],
        )

    def out_start(l, p):
        for di in range(8):
            out_cp(l, p, di).start()

    def out_wait(l, p):
        for di in range(8):
            out_cp(l, p, di).wait()

    iota16 = lax.iota(jnp.int32, 16)
    didx = [iota16 + 16 * k for k in range(4)]

    def transpose(p, q):
        @pl.loop(0, _BW, unroll=8)
        def _per_c(c):
            cb = jnp.full((16,), 0, jnp.int32) + c
            for k in range(4):
                vals = gbufs[p][c, pl.ds(16 * k, 16)]
                plsc.store_scatter(obufs[q], [didx[k], cb], vals)

    for l in range(4):
        gather_cp(l, l).start()
    for l in range(2):
        gather_cp(l, l).wait()
        transpose(l, l % 2)
        out_start(l, l % 2)
        gather_cp(l + 4, l).start()
    for l in range(2, 4):
        gather_cp(l, l).wait()
        out_wait(l - 2, l % 2)
        transpose(l, l % 2)
        out_start(l, l % 2)
        gather_cp(l + 4, l).start()

    @pl.loop(0, (_L - 8) // 4)
    def _steady(i):
        for j in range(4):
            l = 4 * i + 4 + j
            q = j % 2
            gather_cp(l, j).wait()
            out_wait(l - 2, q)
            transpose(j, q)
            out_start(l, q)
            gather_cp(l + 4, j).start()

    for j in range(4):
        l = _L - 4 + j
        q = j % 2
        gather_cp(l, j).wait()
        out_wait(l - 2, q)
        transpose(j, q)
        out_start(l, q)
    for l in range(_L - 2, _L):
        out_wait(l, l % 2)


def kernel(x, w):
    wp = jnp.pad(w, ((0, 0), (0, _D)))
    out5 = _embedding_gather(wp, x.T)
    return out5.transpose(2, 4, 0, 1, 3).reshape(_B, _L, _D)


# speedup vs baseline: 1.0120x; 1.0120x over previous
import functools

import jax
import jax.numpy as jnp
from jax import lax
from jax.experimental import pallas as pl
from jax.experimental.pallas import tpu as pltpu
from jax.experimental.pallas import tpu_sc as plsc

_V = 1000000
_D = 64
_B = 4096
_L = 200
_NC = 2
_NS = 16
_NW = _NC * _NS
_BW = 128

_mesh = plsc.VectorSubcoreMesh(core_axis_name="c", subcore_axis_name="s")


@functools.partial(
    pl.kernel,
    mesh=_mesh,
    compiler_params=pltpu.CompilerParams(
        use_tc_tiling_on_sc=False, needs_layout_passes=False
    ),
    out_type=jax.ShapeDtypeStruct((_L, 8, _NW, 8, 128), jnp.float32),
    scratch_types=[
        pltpu.VMEM((_L, _BW), jnp.int32),
        [pltpu.VMEM((_BW, 2 * _D), jnp.float32) for _ in range(2)],
        [pltpu.VMEM((_D, 129), jnp.float32) for _ in range(2)],
        [pltpu.SemaphoreType.DMA for _ in range(2)],
        [pltpu.SemaphoreType.DMA for _ in range(2)],
    ],
)
def _embedding_gather(w_hbm, idx_hbm, out_hbm, idx_v, gbufs, obufs, gsems, osems):
    wid = lax.axis_index("s") * _NC + lax.axis_index("c")

    pltpu.sync_copy(idx_hbm.at[:, pl.ds(wid * _BW, _BW)], idx_v)

    def gather_cp(l, p):
        return pltpu.make_async_copy(w_hbm.at[idx_v.at[l]], gbufs[p], gsems[p])

    def out_cp(l, p, di):
        return pltpu.make_async_copy(
            obufs[p].at[pl.ds(di * 8, 8), pl.ds(0, 128)],
            out_hbm.at[l, di, wid],
            osems[p],
        )

    def out_start(l, p):
        for di in range(8):
            out_cp(l, p, di).start()

    def out_wait(l, p):
        for di in range(8):
            out_cp(l, p, di).wait()

    iota16 = lax.iota(jnp.int32, 16)
    didx = [iota16 + 16 * k for k in range(4)]

    def transpose(p):
        @pl.loop(0, _BW, unroll=8)
        def _per_c(c):
            cb = jnp.full((16,), 0, jnp.int32) + c
            for k in range(4):
                vals = gbufs[p][c, pl.ds(16 * k, 16)]
                plsc.store_scatter(obufs[p], [didx[k], cb], vals)

    gather_cp(0, 0).start()
    gather_cp(1, 1).start()
    gather_cp(0, 0).wait()
    transpose(0)
    out_start(0, 0)
    gather_cp(2, 0).start()
    gather_cp(1, 1).wait()
    transpose(1)
    out_start(1, 1)
    gather_cp(3, 1).start()

    @pl.loop(0, (_L - 4) // 2)
    def _steady(i):
        for p in range(2):
            l = 2 * i + 2 + p
            gather_cp(l, p).wait()
            out_wait(l - 2, p)
            transpose(p)
            out_start(l, p)
            gather_cp(l + 2, p).start()

    for p in range(2):
        l = _L - 2 + p
        gather_cp(l, p).wait()
        out_wait(l - 2, p)
        transpose(p)
        out_start(l, p)
    for p in range(2):
        out_wait(_L - 2 + p, p)


def kernel(x, w):
    wp = jnp.pad(w, ((0, 0), (0, _D)))
    out5 = _embedding_gather(wp, x.T)
    return out5.transpose(2, 4, 0, 1, 3).reshape(_B, _L, _D)
